# Initial kernel scaffold; baseline (speedup 1.0000x reference)
import jax
import jax.numpy as jnp
from jax.experimental import pallas as pl


def kernel(node_input_src, node_input_dst, batch_dst, edge_src, edge_dst, edge_attr, edge_scalars, W_src, W_dst, b_dst, W1, b1, W2, b2, W_ea, w_alpha, W_v, W_proj, b_proj, ln_w, ln_b, Wf1, bf1, Wf2, bf2):
    raise NotImplementedError("write your pallas kernel here")


# speedup vs baseline: 15.2416x; 15.2416x over previous
import functools

import jax
import jax.numpy as jnp
from jax import lax
from jax.experimental import pallas as pl
from jax.experimental.pallas import tpu as pltpu
from jax.experimental.pallas import tpu_sc as plsc

N = 10000
E = 320000
D = 128
H = 16
DH = 8
DE = 16
FC = 64
DM = 384

NP = 10112
EP = 327680
NW = 32
EPW = EP // NW
GC = 64
NCH = EPW // GC
RPT = NP // 16

f32 = jnp.float32


def _node_msg_body(ns_ref, nd_ref, ws_ref, wd_ref, bd_ref, ms_ref, md_ref):
    ms_ref[...] = jnp.dot(ns_ref[...], ws_ref[...], preferred_element_type=f32)
    md_ref[...] = (
        jnp.dot(nd_ref[...], wd_ref[...], preferred_element_type=f32) + bd_ref[...]
    )


def _node_msgs(ns_p, nd_p, W_src, W_dst, b_dst):
    blk = 1264
    return pl.pallas_call(
        _node_msg_body,
        grid=(NP // blk,),
        in_specs=[
            pl.BlockSpec((blk, D), lambda i: (i, 0)),
            pl.BlockSpec((blk, D), lambda i: (i, 0)),
            pl.BlockSpec((D, D), lambda i: (0, 0)),
            pl.BlockSpec((D, D), lambda i: (0, 0)),
            pl.BlockSpec((1, D), lambda i: (0, 0)),
        ],
        out_specs=[
            pl.BlockSpec((blk, D), lambda i: (i, 0)),
            pl.BlockSpec((blk, D), lambda i: (i, 0)),
        ],
        out_shape=[
            jax.ShapeDtypeStruct((NP, D), f32),
            jax.ShapeDtypeStruct((NP, D), f32),
        ],
    )(ns_p, nd_p, W_src, W_dst, b_dst.reshape(1, D))


def _sc_gather(msrc, mdst, esrc, edst):
    mesh = plsc.VectorSubcoreMesh(core_axis_name="c", subcore_axis_name="s")

    @functools.partial(
        pl.kernel,
        mesh=mesh,
        out_type=(
            jax.ShapeDtypeStruct((EP, D), f32),
            jax.ShapeDtypeStruct((EP, D), f32),
        ),
        scratch_types=[
            pltpu.VMEM((GC,), jnp.int32),
            pltpu.VMEM((GC,), jnp.int32),
            pltpu.VMEM((GC, D), f32),
            pltpu.VMEM((GC, D), f32),
            pltpu.SemaphoreType.DMA,
        ],
    )
    def k(msrc_h, mdst_h, esrc_h, edst_h, outs_h, outd_h, ixs, ixd, ra, rb, sem):
        c = lax.axis_index("c")
        s = lax.axis_index("s")
        base0 = (s * 2 + c) * EPW

        def body(i, carry):
            base = base0 + i * GC
            pltpu.sync_copy(esrc_h.at[pl.ds(base, GC)], ixs)
            pltpu.sync_copy(edst_h.at[pl.ds(base, GC)], ixd)
            cp1 = pltpu.async_copy(msrc_h.at[ixs], ra, sem)
            cp1.wait()
            cp2 = pltpu.async_copy(mdst_h.at[ixd], rb, sem)
            cp2.wait()
            pltpu.sync_copy(ra, outs_h.at[pl.ds(base, GC), :])
            pltpu.sync_copy(rb, outd_h.at[pl.ds(base, GC), :])
            return carry

        lax.fori_loop(0, NCH, body, 0)

    return k(msrc, mdst, esrc, edst)


def _edge_body(ms_ref, md_ref, es_ref, ea_ref, w1_ref, b1_ref, w2_ref, b2_ref,
               wea_ref, aa_ref, rr_ref, wv_ref, ev_ref, ex_ref):
    msg = ms_ref[...] + md_ref[...]
    h = jax.nn.silu(
        jnp.dot(es_ref[...], w1_ref[...], preferred_element_type=f32) + b1_ref[...]
    )
    w = jnp.dot(h, w2_ref[...], preferred_element_type=f32) + b2_ref[...]
    dtp = msg * w + jnp.dot(ea_ref[...], wea_ref[...], preferred_element_type=f32)
    a = jnp.where(dtp >= 0, dtp, 0.2 * dtp)
    exv = jnp.exp(jnp.dot(a, aa_ref[...], preferred_element_type=f32))
    v = jnp.dot(dtp, wv_ref[...], preferred_element_type=f32)
    exw = jnp.dot(exv, rr_ref[...], preferred_element_type=f32)
    ev_ref[...] = v * exw
    ex_ref[...] = exw


def _edge_pipeline(msgg_s, msgg_d, es_p, ea_p, W1, b1, W2, b2, W_ea, Aa, Rr, W_v):
    bt = 512
    return pl.pallas_call(
        _edge_body,
        grid=(EP // bt,),
        in_specs=[
            pl.BlockSpec((bt, D), lambda i: (i, 0)),
            pl.BlockSpec((bt, D), lambda i: (i, 0)),
            pl.BlockSpec((bt, FC), lambda i: (i, 0)),
            pl.BlockSpec((bt, DE), lambda i: (i, 0)),
            pl.BlockSpec((FC, FC), lambda i: (0, 0)),
            pl.BlockSpec((1, FC), lambda i: (0, 0)),
            pl.BlockSpec((FC, D), lambda i: (0, 0)),
            pl.BlockSpec((1, D), lambda i: (0, 0)),
            pl.BlockSpec((DE, D), lambda i: (0, 0)),
            pl.BlockSpec((D, H), lambda i: (0, 0)),
            pl.BlockSpec((H, D), lambda i: (0, 0)),
            pl.BlockSpec((D, D), lambda i: (0, 0)),
        ],
        out_specs=[
            pl.BlockSpec((bt, D), lambda i: (i, 0)),
            pl.BlockSpec((bt, D), lambda i: (i, 0)),
        ],
        out_shape=[
            jax.ShapeDtypeStruct((EP, D), f32),
            jax.ShapeDtypeStruct((EP, D), f32),
        ],
    )(msgg_s, msgg_d, es_p, ea_p, W1, b1.reshape(1, FC), W2, b2.reshape(1, D),
      W_ea, Aa, Rr, W_v)


def _sc_scatter128(vals, edst, zS):
    mesh = plsc.VectorSubcoreMesh(core_axis_name="c", subcore_axis_name="s")

    @functools.partial(
        pl.kernel,
        mesh=mesh,
        out_type=jax.ShapeDtypeStruct((2, NP, D), f32),
        scratch_types=[
            pltpu.VMEM((GC,), jnp.int32),
            pltpu.VMEM((GC, D), f32),
            pltpu.VMEM_SHARED((NP, D), f32),
            pltpu.SemaphoreType.DMA,
        ],
    )
    def k(val_h, edst_h, zs_h, out_h, ix, vv, ssh, sem):
        c = lax.axis_index("c")
        s = lax.axis_index("s")
        r0 = s * RPT
        pltpu.sync_copy(zs_h.at[pl.ds(r0, RPT), :], ssh.at[pl.ds(r0, RPT), :])
        plsc.subcore_barrier()

        base0 = c * (EP // 2) + s * EPW

        def body(i, carry):
            base = base0 + i * GC
            pltpu.sync_copy(edst_h.at[pl.ds(base, GC)], ix)
            pltpu.sync_copy(val_h.at[pl.ds(base, GC), :], vv)
            pltpu.sync_copy(vv, ssh.at[ix], add=True)
            return carry

        lax.fori_loop(0, NCH, body, 0)
        plsc.subcore_barrier()
        pltpu.sync_copy(ssh.at[pl.ds(r0, RPT), :], out_h.at[c, pl.ds(r0, RPT), :])

    return k(vals, edst, zS)


def _final_body(s_ref, dn_ref, nd_ref, wp_ref, bp_ref, lnw_ref, lnb_ref,
                wf1_ref, bf1_ref, wf2_ref, bf2_ref, out_ref):
    sv = s_ref[0] + s_ref[1]
    denw = dn_ref[0] + dn_ref[1]
    agg = sv / (denw + 1e-9)
    nf = jnp.dot(agg, wp_ref[...], preferred_element_type=f32) + bp_ref[...]
    node_out = nd_ref[...] + nf
    mu = jnp.mean(node_out, axis=-1, keepdims=True)
    xc = node_out - mu
    var = jnp.mean(xc * xc, axis=-1, keepdims=True)
    nrm = xc / jnp.sqrt(var + 1e-5) * lnw_ref[...] + lnb_ref[...]
    h2 = jax.nn.silu(
        jnp.dot(nrm, wf1_ref[...], preferred_element_type=f32) + bf1_ref[...]
    )
    nf2 = jnp.dot(h2, wf2_ref[...], preferred_element_type=f32) + bf2_ref[...]
    out_ref[...] = node_out + nf2


def _node_final(outS, outDen, node_input_dst, W_proj, b_proj, ln_w, ln_b,
                Wf1, bf1, Wf2, bf2):
    bn = 1000
    return pl.pallas_call(
        _final_body,
        grid=(N // bn,),
        in_specs=[
            pl.BlockSpec((2, bn, D), lambda i: (0, i, 0)),
            pl.BlockSpec((2, bn, D), lambda i: (0, i, 0)),
            pl.BlockSpec((bn, D), lambda i: (i, 0)),
            pl.BlockSpec((D, D), lambda i: (0, 0)),
            pl.BlockSpec((1, D), lambda i: (0, 0)),
            pl.BlockSpec((1, D), lambda i: (0, 0)),
            pl.BlockSpec((1, D), lambda i: (0, 0)),
            pl.BlockSpec((D, DM), lambda i: (0, 0)),
            pl.BlockSpec((1, DM), lambda i: (0, 0)),
            pl.BlockSpec((DM, D), lambda i: (0, 0)),
            pl.BlockSpec((1, D), lambda i: (0, 0)),
        ],
        out_specs=pl.BlockSpec((bn, D), lambda i: (i, 0)),
        out_shape=jax.ShapeDtypeStruct((N, D), f32),
    )(outS, outDen, node_input_dst, W_proj, b_proj.reshape(1, D),
      ln_w.reshape(1, D), ln_b.reshape(1, D), Wf1, bf1.reshape(1, DM),
      Wf2, bf2.reshape(1, D))


def kernel(node_input_src, node_input_dst, batch_dst, edge_src, edge_dst,
           edge_attr, edge_scalars, W_src, W_dst, b_dst, W1, b1, W2, b2, W_ea,
           w_alpha, W_v, W_proj, b_proj, ln_w, ln_b, Wf1, bf1, Wf2, bf2):
    ns_p = jnp.zeros((NP, D), f32).at[:N].set(node_input_src)
    nd_p = jnp.zeros((NP, D), f32).at[:N].set(node_input_dst)
    esrc_p = jnp.concatenate([edge_src, jnp.zeros((EP - E,), jnp.int32)])
    edst_p = jnp.concatenate([edge_dst, jnp.full((EP - E,), N, jnp.int32)])
    es_p = jnp.zeros((EP, FC), f32).at[:E].set(edge_scalars)
    ea_p = jnp.zeros((EP, DE), f32).at[:E].set(edge_attr)
    eye16 = jnp.eye(H, dtype=f32)
    Rr = jnp.repeat(eye16, DH, axis=1)
    Aa = (w_alpha[:, :, None] * eye16[:, None, :]).reshape(D, H)

    msg_src, msg_dst = _node_msgs(ns_p, nd_p, W_src, W_dst, b_dst)
    msgg_s, msgg_d = _sc_gather(msg_src, msg_dst, esrc_p, edst_p)
    ev, exw = _edge_pipeline(msgg_s, msgg_d, es_p, ea_p, W1, b1, W2, b2, W_ea,
                             Aa, Rr, W_v)
    zS = jnp.zeros((NP, D), f32)
    outS = _sc_scatter128(ev, edst_p, zS)
    outDen = _sc_scatter128(exw, edst_p, zS)
    return _node_final(outS, outDen, node_input_dst, W_proj, b_proj, ln_w, ln_b,
                       Wf1, bf1, Wf2, bf2)


# speedup vs baseline: 19.0831x; 1.2520x over previous
import functools

import jax
import jax.numpy as jnp
from jax import lax
from jax.experimental import pallas as pl
from jax.experimental.pallas import tpu as pltpu
from jax.experimental.pallas import tpu_sc as plsc

N = 10000
E = 320000
D = 128
H = 16
DH = 8
DE = 16
FC = 64
DM = 384

NP = 10112
EP = 327680
NW = 32
EPW = EP // NW
GC = 128
NCH = EPW // GC
RPT = NP // 16

f32 = jnp.float32


def _node_msg_body(ns_ref, nd_ref, ws_ref, wd_ref, bd_ref, ms_ref, md_ref):
    ms_ref[...] = jnp.dot(ns_ref[...], ws_ref[...], preferred_element_type=f32)
    md_ref[...] = (
        jnp.dot(nd_ref[...], wd_ref[...], preferred_element_type=f32) + bd_ref[...]
    )


def _node_msgs(ns_p, nd_p, W_src, W_dst, b_dst):
    blk = 1264
    return pl.pallas_call(
        _node_msg_body,
        grid=(NP // blk,),
        in_specs=[
            pl.BlockSpec((blk, D), lambda i: (i, 0)),
            pl.BlockSpec((blk, D), lambda i: (i, 0)),
            pl.BlockSpec((D, D), lambda i: (0, 0)),
            pl.BlockSpec((D, D), lambda i: (0, 0)),
            pl.BlockSpec((1, D), lambda i: (0, 0)),
        ],
        out_specs=[
            pl.BlockSpec((blk, D), lambda i: (i, 0)),
            pl.BlockSpec((blk, D), lambda i: (i, 0)),
        ],
        out_shape=[
            jax.ShapeDtypeStruct((NP, D), f32),
            jax.ShapeDtypeStruct((NP, D), f32),
        ],
    )(ns_p, nd_p, W_src, W_dst, b_dst.reshape(1, D))


def _sc_gather(msrc, mdst, esrc2d, edst2d):
    mesh = plsc.VectorSubcoreMesh(core_axis_name="c", subcore_axis_name="s")

    @functools.partial(
        pl.kernel,
        mesh=mesh,
        out_type=(
            jax.ShapeDtypeStruct((EP, D), f32),
            jax.ShapeDtypeStruct((EP, D), f32),
        ),
        scratch_types=[
            pltpu.VMEM((NCH, GC), jnp.int32),
            pltpu.VMEM((NCH, GC), jnp.int32),
            pltpu.VMEM((2, GC, D), f32),
            pltpu.VMEM((2, GC, D), f32),
            pltpu.SemaphoreType.DMA,
            pltpu.SemaphoreType.DMA,
            pltpu.SemaphoreType.DMA,
            pltpu.SemaphoreType.DMA,
        ],
    )
    def k(msrc_h, mdst_h, esrc_h, edst_h, outs_h, outd_h,
          ixs, ixd, ra, rb, g0, g1, w0, w1):
        c = lax.axis_index("c")
        s = lax.axis_index("s")
        wid = s * 2 + c
        chunk0 = wid * NCH
        base0 = wid * EPW
        gsem = (g0, g1)
        wsem = (w0, w1)
        pltpu.sync_copy(esrc_h.at[pl.ds(chunk0, NCH), :], ixs)
        pltpu.sync_copy(edst_h.at[pl.ds(chunk0, NCH), :], ixd)

        def outer(t, carry):
            i0 = t * 2

            @pl.when(t > 0)
            def _():
                for b in range(2):
                    pltpu.make_async_copy(
                        ra.at[b], outs_h.at[pl.ds(0, GC), :], wsem[b]).wait()
                    pltpu.make_async_copy(
                        rb.at[b], outd_h.at[pl.ds(0, GC), :], wsem[b]).wait()

            for b in range(2):
                i = i0 + b
                pltpu.make_async_copy(msrc_h.at[ixs.at[i]], ra.at[b], gsem[b]).start()
                pltpu.make_async_copy(mdst_h.at[ixd.at[i]], rb.at[b], gsem[b]).start()

            for b in range(2):
                i = i0 + b
                base = base0 + i * GC
                pltpu.make_async_copy(msrc_h.at[ixs.at[i]], ra.at[b], gsem[b]).wait()
                pltpu.make_async_copy(mdst_h.at[ixd.at[i]], rb.at[b], gsem[b]).wait()
                pltpu.make_async_copy(ra.at[b], outs_h.at[pl.ds(base, GC), :], wsem[b]).start()
                pltpu.make_async_copy(rb.at[b], outd_h.at[pl.ds(base, GC), :], wsem[b]).start()
            return carry

        lax.fori_loop(0, NCH // 2, outer, 0)
        for b in range(2):
            pltpu.make_async_copy(ra.at[b], outs_h.at[pl.ds(0, GC), :], wsem[b]).wait()
            pltpu.make_async_copy(rb.at[b], outd_h.at[pl.ds(0, GC), :], wsem[b]).wait()

    return k(msrc, mdst, esrc2d, edst2d)


def _edge_body(ms_ref, md_ref, es_ref, ea_ref, w1_ref, b1_ref, w2_ref, b2_ref,
               wea_ref, aa_ref, rr_ref, wv_ref, ev_ref, ex_ref):
    msg = ms_ref[...] + md_ref[...]
    h = jax.nn.silu(
        jnp.dot(es_ref[...], w1_ref[...], preferred_element_type=f32) + b1_ref[...]
    )
    w = jnp.dot(h, w2_ref[...], preferred_element_type=f32) + b2_ref[...]
    dtp = msg * w + jnp.dot(ea_ref[...], wea_ref[...], preferred_element_type=f32)
    a = jnp.where(dtp >= 0, dtp, 0.2 * dtp)
    exv = jnp.exp(jnp.dot(a, aa_ref[...], preferred_element_type=f32))
    v = jnp.dot(dtp, wv_ref[...], preferred_element_type=f32)
    exw = jnp.dot(exv, rr_ref[...], preferred_element_type=f32)
    ev_ref[...] = v * exw
    ex_ref[...] = exw


def _edge_pipeline(msgg_s, msgg_d, es_p, ea_p, W1, b1, W2, b2, W_ea, Aa, Rr, W_v):
    bt = 512
    return pl.pallas_call(
        _edge_body,
        grid=(EP // bt,),
        in_specs=[
            pl.BlockSpec((bt, D), lambda i: (i, 0)),
            pl.BlockSpec((bt, D), lambda i: (i, 0)),
            pl.BlockSpec((bt, FC), lambda i: (i, 0)),
            pl.BlockSpec((bt, DE), lambda i: (i, 0)),
            pl.BlockSpec((FC, FC), lambda i: (0, 0)),
            pl.BlockSpec((1, FC), lambda i: (0, 0)),
            pl.BlockSpec((FC, D), lambda i: (0, 0)),
            pl.BlockSpec((1, D), lambda i: (0, 0)),
            pl.BlockSpec((DE, D), lambda i: (0, 0)),
            pl.BlockSpec((D, H), lambda i: (0, 0)),
            pl.BlockSpec((H, D), lambda i: (0, 0)),
            pl.BlockSpec((D, D), lambda i: (0, 0)),
        ],
        out_specs=[
            pl.BlockSpec((bt, D), lambda i: (i, 0)),
            pl.BlockSpec((bt, D), lambda i: (i, 0)),
        ],
        out_shape=[
            jax.ShapeDtypeStruct((EP, D), f32),
            jax.ShapeDtypeStruct((EP, D), f32),
        ],
    )(msgg_s, msgg_d, es_p, ea_p, W1, b1.reshape(1, FC), W2, b2.reshape(1, D),
      W_ea, Aa, Rr, W_v)


def _sc_scatter128(vals, edst, zS):
    mesh = plsc.VectorSubcoreMesh(core_axis_name="c", subcore_axis_name="s")

    @functools.partial(
        pl.kernel,
        mesh=mesh,
        out_type=jax.ShapeDtypeStruct((2, NP, D), f32),
        scratch_types=[
            pltpu.VMEM((GC,), jnp.int32),
            pltpu.VMEM((GC, D), f32),
            pltpu.VMEM_SHARED((NP, D), f32),
            pltpu.SemaphoreType.DMA,
        ],
    )
    def k(val_h, edst_h, zs_h, out_h, ix, vv, ssh, sem):
        c = lax.axis_index("c")
        s = lax.axis_index("s")
        r0 = s * RPT
        pltpu.sync_copy(zs_h.at[pl.ds(r0, RPT), :], ssh.at[pl.ds(r0, RPT), :])
        plsc.subcore_barrier()

        base0 = c * (EP // 2) + s * EPW

        def body(i, carry):
            base = base0 + i * GC
            pltpu.sync_copy(edst_h.at[pl.ds(base, GC)], ix)
            pltpu.sync_copy(val_h.at[pl.ds(base, GC), :], vv)
            pltpu.sync_copy(vv, ssh.at[ix], add=True)
            return carry

        lax.fori_loop(0, NCH, body, 0)
        plsc.subcore_barrier()
        pltpu.sync_copy(ssh.at[pl.ds(r0, RPT), :], out_h.at[c, pl.ds(r0, RPT), :])

    return k(vals, edst, zS)


def _final_body(s_ref, dn_ref, nd_ref, wp_ref, bp_ref, lnw_ref, lnb_ref,
                wf1_ref, bf1_ref, wf2_ref, bf2_ref, out_ref):
    sv = s_ref[0] + s_ref[1]
    denw = dn_ref[0] + dn_ref[1]
    agg = sv / (denw + 1e-9)
    nf = jnp.dot(agg, wp_ref[...], preferred_element_type=f32) + bp_ref[...]
    node_out = nd_ref[...] + nf
    mu = jnp.mean(node_out, axis=-1, keepdims=True)
    xc = node_out - mu
    var = jnp.mean(xc * xc, axis=-1, keepdims=True)
    nrm = xc / jnp.sqrt(var + 1e-5) * lnw_ref[...] + lnb_ref[...]
    h2 = jax.nn.silu(
        jnp.dot(nrm, wf1_ref[...], preferred_element_type=f32) + bf1_ref[...]
    )
    nf2 = jnp.dot(h2, wf2_ref[...], preferred_element_type=f32) + bf2_ref[...]
    out_ref[...] = node_out + nf2


def _node_final(outS, outDen, node_input_dst, W_proj, b_proj, ln_w, ln_b,
                Wf1, bf1, Wf2, bf2):
    bn = 1000
    return pl.pallas_call(
        _final_body,
        grid=(N // bn,),
        in_specs=[
            pl.BlockSpec((2, bn, D), lambda i: (0, i, 0)),
            pl.BlockSpec((2, bn, D), lambda i: (0, i, 0)),
            pl.BlockSpec((bn, D), lambda i: (i, 0)),
            pl.BlockSpec((D, D), lambda i: (0, 0)),
            pl.BlockSpec((1, D), lambda i: (0, 0)),
            pl.BlockSpec((1, D), lambda i: (0, 0)),
            pl.BlockSpec((1, D), lambda i: (0, 0)),
            pl.BlockSpec((D, DM), lambda i: (0, 0)),
            pl.BlockSpec((1, DM), lambda i: (0, 0)),
            pl.BlockSpec((DM, D), lambda i: (0, 0)),
            pl.BlockSpec((1, D), lambda i: (0, 0)),
        ],
        out_specs=pl.BlockSpec((bn, D), lambda i: (i, 0)),
        out_shape=jax.ShapeDtypeStruct((N, D), f32),
    )(outS, outDen, node_input_dst, W_proj, b_proj.reshape(1, D),
      ln_w.reshape(1, D), ln_b.reshape(1, D), Wf1, bf1.reshape(1, DM),
      Wf2, bf2.reshape(1, D))


def kernel(node_input_src, node_input_dst, batch_dst, edge_src, edge_dst,
           edge_attr, edge_scalars, W_src, W_dst, b_dst, W1, b1, W2, b2, W_ea,
           w_alpha, W_v, W_proj, b_proj, ln_w, ln_b, Wf1, bf1, Wf2, bf2):
    ns_p = jnp.zeros((NP, D), f32).at[:N].set(node_input_src)
    nd_p = jnp.zeros((NP, D), f32).at[:N].set(node_input_dst)
    esrc_p = jnp.concatenate([edge_src, jnp.zeros((EP - E,), jnp.int32)])
    edst_p = jnp.concatenate([edge_dst, jnp.full((EP - E,), N, jnp.int32)])
    es_p = jnp.zeros((EP, FC), f32).at[:E].set(edge_scalars)
    ea_p = jnp.zeros((EP, DE), f32).at[:E].set(edge_attr)
    eye16 = jnp.eye(H, dtype=f32)
    Rr = jnp.repeat(eye16, DH, axis=1)
    Aa = (w_alpha[:, :, None] * eye16[:, None, :]).reshape(D, H)

    msg_src, msg_dst = _node_msgs(ns_p, nd_p, W_src, W_dst, b_dst)
    msgg_s, msgg_d = _sc_gather(msg_src, msg_dst, esrc_p.reshape(EP // GC, GC),
                                edst_p.reshape(EP // GC, GC))
    ev, exw = _edge_pipeline(msgg_s, msgg_d, es_p, ea_p, W1, b1, W2, b2, W_ea,
                             Aa, Rr, W_v)
    zS = jnp.zeros((NP, D), f32)
    outS = _sc_scatter128(ev, edst_p, zS)
    outDen = _sc_scatter128(exw, edst_p, zS)
    return _node_final(outS, outDen, node_input_dst, W_proj, b_proj, ln_w, ln_b,
                       Wf1, bf1, Wf2, bf2)


# speedup vs baseline: 20.0061x; 1.0484x over previous
import functools

import jax
import jax.numpy as jnp
from jax import lax
from jax.experimental import pallas as pl
from jax.experimental.pallas import tpu as pltpu
from jax.experimental.pallas import tpu_sc as plsc

N = 10000
E = 320000
D = 128
H = 16
DH = 8
DE = 16
FC = 64
DM = 384

NP = 10112
EP = 327680
NW = 32
EPW = EP // NW
GC = 128
NCH = EPW // GC
RPT = NP // 16

f32 = jnp.float32


def _node_msg_body(ns_ref, nd_ref, ws_ref, wd_ref, bd_ref, ms_ref, md_ref):
    ms_ref[...] = jnp.dot(ns_ref[...], ws_ref[...], preferred_element_type=f32)
    md_ref[...] = (
        jnp.dot(nd_ref[...], wd_ref[...], preferred_element_type=f32) + bd_ref[...]
    )


def _node_msgs(ns_p, nd_p, W_src, W_dst, b_dst):
    blk = 1264
    return pl.pallas_call(
        _node_msg_body,
        grid=(NP // blk,),
        in_specs=[
            pl.BlockSpec((blk, D), lambda i: (i, 0)),
            pl.BlockSpec((blk, D), lambda i: (i, 0)),
            pl.BlockSpec((D, D), lambda i: (0, 0)),
            pl.BlockSpec((D, D), lambda i: (0, 0)),
            pl.BlockSpec((1, D), lambda i: (0, 0)),
        ],
        out_specs=[
            pl.BlockSpec((blk, D), lambda i: (i, 0)),
            pl.BlockSpec((blk, D), lambda i: (i, 0)),
        ],
        out_shape=[
            jax.ShapeDtypeStruct((NP, D), f32),
            jax.ShapeDtypeStruct((NP, D), f32),
        ],
    )(ns_p, nd_p, W_src, W_dst, b_dst.reshape(1, D))


def _sc_gather(msrc, mdst, esrc2d, edst2d):
    mesh = plsc.VectorSubcoreMesh(core_axis_name="c", subcore_axis_name="s")

    @functools.partial(
        pl.kernel,
        mesh=mesh,
        out_type=(
            jax.ShapeDtypeStruct((EP, D), f32),
            jax.ShapeDtypeStruct((EP, D), f32),
        ),
        scratch_types=[
            pltpu.VMEM((NCH, GC), jnp.int32),
            pltpu.VMEM((NCH, GC), jnp.int32),
            pltpu.VMEM((2, GC, D), f32),
            pltpu.VMEM((2, GC, D), f32),
            pltpu.SemaphoreType.DMA,
            pltpu.SemaphoreType.DMA,
            pltpu.SemaphoreType.DMA,
            pltpu.SemaphoreType.DMA,
        ],
    )
    def k(msrc_h, mdst_h, esrc_h, edst_h, outs_h, outd_h,
          ixs, ixd, ra, rb, g0, g1, w0, w1):
        c = lax.axis_index("c")
        s = lax.axis_index("s")
        wid = s * 2 + c
        chunk0 = wid * NCH
        base0 = wid * EPW
        gsem = (g0, g1)
        wsem = (w0, w1)
        pltpu.sync_copy(esrc_h.at[pl.ds(chunk0, NCH), :], ixs)
        pltpu.sync_copy(edst_h.at[pl.ds(chunk0, NCH), :], ixd)

        def outer(t, carry):
            i0 = t * 2

            @pl.when(t > 0)
            def _():
                for b in range(2):
                    pltpu.make_async_copy(
                        ra.at[b], outs_h.at[pl.ds(0, GC), :], wsem[b]).wait()
                    pltpu.make_async_copy(
                        rb.at[b], outd_h.at[pl.ds(0, GC), :], wsem[b]).wait()

            for b in range(2):
                i = i0 + b
                pltpu.make_async_copy(msrc_h.at[ixs.at[i]], ra.at[b], gsem[b]).start()
                pltpu.make_async_copy(mdst_h.at[ixd.at[i]], rb.at[b], gsem[b]).start()

            for b in range(2):
                i = i0 + b
                base = base0 + i * GC
                pltpu.make_async_copy(msrc_h.at[ixs.at[i]], ra.at[b], gsem[b]).wait()
                pltpu.make_async_copy(mdst_h.at[ixd.at[i]], rb.at[b], gsem[b]).wait()
                pltpu.make_async_copy(ra.at[b], outs_h.at[pl.ds(base, GC), :], wsem[b]).start()
                pltpu.make_async_copy(rb.at[b], outd_h.at[pl.ds(base, GC), :], wsem[b]).start()
            return carry

        lax.fori_loop(0, NCH // 2, outer, 0)
        for b in range(2):
            pltpu.make_async_copy(ra.at[b], outs_h.at[pl.ds(0, GC), :], wsem[b]).wait()
            pltpu.make_async_copy(rb.at[b], outd_h.at[pl.ds(0, GC), :], wsem[b]).wait()

    return k(msrc, mdst, esrc2d, edst2d)


def _edge_body(ms_ref, md_ref, es_ref, ea_ref, w1_ref, b1_ref, w2_ref, b2_ref,
               wea_ref, aa_ref, rr_ref, wv_ref, ev_ref, ex_ref):
    msg = ms_ref[...] + md_ref[...]
    h = jax.nn.silu(
        jnp.dot(es_ref[...], w1_ref[...], preferred_element_type=f32) + b1_ref[...]
    )
    w = jnp.dot(h, w2_ref[...], preferred_element_type=f32) + b2_ref[...]
    dtp = msg * w + jnp.dot(ea_ref[...], wea_ref[...], preferred_element_type=f32)
    a = jnp.where(dtp >= 0, dtp, 0.2 * dtp)
    exv = jnp.exp(jnp.dot(a, aa_ref[...], preferred_element_type=f32))
    v = jnp.dot(dtp, wv_ref[...], preferred_element_type=f32)
    exw = jnp.dot(exv, rr_ref[...], preferred_element_type=f32)
    ev_ref[...] = v * exw
    ex_ref[...] = exw


def _edge_pipeline(msgg_s, msgg_d, es_p, ea_p, W1, b1, W2, b2, W_ea, Aa, Rr, W_v):
    bt = 512
    return pl.pallas_call(
        _edge_body,
        grid=(EP // bt,),
        in_specs=[
            pl.BlockSpec((bt, D), lambda i: (i, 0)),
            pl.BlockSpec((bt, D), lambda i: (i, 0)),
            pl.BlockSpec((bt, FC), lambda i: (i, 0)),
            pl.BlockSpec((bt, DE), lambda i: (i, 0)),
            pl.BlockSpec((FC, FC), lambda i: (0, 0)),
            pl.BlockSpec((1, FC), lambda i: (0, 0)),
            pl.BlockSpec((FC, D), lambda i: (0, 0)),
            pl.BlockSpec((1, D), lambda i: (0, 0)),
            pl.BlockSpec((DE, D), lambda i: (0, 0)),
            pl.BlockSpec((D, H), lambda i: (0, 0)),
            pl.BlockSpec((H, D), lambda i: (0, 0)),
            pl.BlockSpec((D, D), lambda i: (0, 0)),
        ],
        out_specs=[
            pl.BlockSpec((bt, D), lambda i: (i, 0)),
            pl.BlockSpec((bt, D), lambda i: (i, 0)),
        ],
        out_shape=[
            jax.ShapeDtypeStruct((EP, D), f32),
            jax.ShapeDtypeStruct((EP, D), f32),
        ],
    )(msgg_s, msgg_d, es_p, ea_p, W1, b1.reshape(1, FC), W2, b2.reshape(1, D),
      W_ea, Aa, Rr, W_v)


def _sc_scatter128(vals, edst2d, zS):
    mesh = plsc.VectorSubcoreMesh(core_axis_name="c", subcore_axis_name="s")

    @functools.partial(
        pl.kernel,
        mesh=mesh,
        out_type=jax.ShapeDtypeStruct((2, NP, D), f32),
        scratch_types=[
            pltpu.VMEM((NCH, GC), jnp.int32),
            pltpu.VMEM((2, GC, D), f32),
            pltpu.VMEM_SHARED((NP, D), f32),
            pltpu.SemaphoreType.DMA,
            pltpu.SemaphoreType.DMA,
        ],
    )
    def k(val_h, edst_h, zs_h, out_h, ix, vv, ssh, r0sem, r1sem):
        c = lax.axis_index("c")
        s = lax.axis_index("s")
        rsem = (r0sem, r1sem)
        r0 = s * RPT
        pltpu.sync_copy(zs_h.at[pl.ds(r0, RPT), :], ssh.at[pl.ds(r0, RPT), :])

        base0 = c * (EP // 2) + s * EPW
        chunk0 = c * (EP // (2 * GC)) + s * NCH
        pltpu.sync_copy(edst_h.at[pl.ds(chunk0, NCH), :], ix)
        plsc.subcore_barrier()

        def outer(t, carry):
            i0 = t * 2
            for b in range(2):
                i = i0 + b
                base = base0 + i * GC
                pltpu.make_async_copy(
                    val_h.at[pl.ds(base, GC), :], vv.at[b], rsem[b]).start()
            for b in range(2):
                i = i0 + b
                base = base0 + i * GC
                pltpu.make_async_copy(
                    val_h.at[pl.ds(base, GC), :], vv.at[b], rsem[b]).wait()
                pltpu.sync_copy(vv.at[b], ssh.at[ix.at[i]], add=True)
            return carry

        lax.fori_loop(0, NCH // 2, outer, 0)
        plsc.subcore_barrier()
        pltpu.sync_copy(ssh.at[pl.ds(r0, RPT), :], out_h.at[c, pl.ds(r0, RPT), :])

    return k(vals, edst2d, zS)


def _final_body(s_ref, dn_ref, nd_ref, wp_ref, bp_ref, lnw_ref, lnb_ref,
                wf1_ref, bf1_ref, wf2_ref, bf2_ref, out_ref):
    sv = s_ref[0] + s_ref[1]
    denw = dn_ref[0] + dn_ref[1]
    agg = sv / (denw + 1e-9)
    nf = jnp.dot(agg, wp_ref[...], preferred_element_type=f32) + bp_ref[...]
    node_out = nd_ref[...] + nf
    mu = jnp.mean(node_out, axis=-1, keepdims=True)
    xc = node_out - mu
    var = jnp.mean(xc * xc, axis=-1, keepdims=True)
    nrm = xc / jnp.sqrt(var + 1e-5) * lnw_ref[...] + lnb_ref[...]
    h2 = jax.nn.silu(
        jnp.dot(nrm, wf1_ref[...], preferred_element_type=f32) + bf1_ref[...]
    )
    nf2 = jnp.dot(h2, wf2_ref[...], preferred_element_type=f32) + bf2_ref[...]
    out_ref[...] = node_out + nf2


def _node_final(outS, outDen, node_input_dst, W_proj, b_proj, ln_w, ln_b,
                Wf1, bf1, Wf2, bf2):
    bn = 1000
    return pl.pallas_call(
        _final_body,
        grid=(N // bn,),
        in_specs=[
            pl.BlockSpec((2, bn, D), lambda i: (0, i, 0)),
            pl.BlockSpec((2, bn, D), lambda i: (0, i, 0)),
            pl.BlockSpec((bn, D), lambda i: (i, 0)),
            pl.BlockSpec((D, D), lambda i: (0, 0)),
            pl.BlockSpec((1, D), lambda i: (0, 0)),
            pl.BlockSpec((1, D), lambda i: (0, 0)),
            pl.BlockSpec((1, D), lambda i: (0, 0)),
            pl.BlockSpec((D, DM), lambda i: (0, 0)),
            pl.BlockSpec((1, DM), lambda i: (0, 0)),
            pl.BlockSpec((DM, D), lambda i: (0, 0)),
            pl.BlockSpec((1, D), lambda i: (0, 0)),
        ],
        out_specs=pl.BlockSpec((bn, D), lambda i: (i, 0)),
        out_shape=jax.ShapeDtypeStruct((N, D), f32),
    )(outS, outDen, node_input_dst, W_proj, b_proj.reshape(1, D),
      ln_w.reshape(1, D), ln_b.reshape(1, D), Wf1, bf1.reshape(1, DM),
      Wf2, bf2.reshape(1, D))


def kernel(node_input_src, node_input_dst, batch_dst, edge_src, edge_dst,
           edge_attr, edge_scalars, W_src, W_dst, b_dst, W1, b1, W2, b2, W_ea,
           w_alpha, W_v, W_proj, b_proj, ln_w, ln_b, Wf1, bf1, Wf2, bf2):
    ns_p = jnp.zeros((NP, D), f32).at[:N].set(node_input_src)
    nd_p = jnp.zeros((NP, D), f32).at[:N].set(node_input_dst)
    esrc_p = jnp.concatenate([edge_src, jnp.zeros((EP - E,), jnp.int32)])
    edst_p = jnp.concatenate([edge_dst, jnp.full((EP - E,), N, jnp.int32)])
    es_p = jnp.zeros((EP, FC), f32).at[:E].set(edge_scalars)
    ea_p = jnp.zeros((EP, DE), f32).at[:E].set(edge_attr)
    eye16 = jnp.eye(H, dtype=f32)
    Rr = jnp.repeat(eye16, DH, axis=1)
    Aa = (w_alpha[:, :, None] * eye16[:, None, :]).reshape(D, H)

    msg_src, msg_dst = _node_msgs(ns_p, nd_p, W_src, W_dst, b_dst)
    msgg_s, msgg_d = _sc_gather(msg_src, msg_dst, esrc_p.reshape(EP // GC, GC),
                                edst_p.reshape(EP // GC, GC))
    ev, exw = _edge_pipeline(msgg_s, msgg_d, es_p, ea_p, W1, b1, W2, b2, W_ea,
                             Aa, Rr, W_v)
    zS = jnp.zeros((NP, D), f32)
    edst2d = edst_p.reshape(EP // GC, GC)
    outS = _sc_scatter128(ev, edst2d, zS)
    outDen = _sc_scatter128(exw, edst2d, zS)
    return _node_final(outS, outDen, node_input_dst, W_proj, b_proj, ln_w, ln_b,
                       Wf1, bf1, Wf2, bf2)


# speedup vs baseline: 22.9987x; 1.1496x over previous
import functools

import jax
import jax.numpy as jnp
from jax import lax
from jax.experimental import pallas as pl
from jax.experimental.pallas import tpu as pltpu
from jax.experimental.pallas import tpu_sc as plsc

N = 10000
E = 320000
D = 128
H = 16
DH = 8
DE = 16
FC = 64
DM = 384

NP = 10112
EP = 327680
NW = 32
EPW = EP // NW
GC = 128
NCH = EPW // GC
RPT = NP // 16

f32 = jnp.float32


def _node_msg_body(ns_ref, nd_ref, ws_ref, wd_ref, bd_ref, ms_ref, md_ref):
    ms_ref[...] = jnp.dot(ns_ref[...], ws_ref[...], preferred_element_type=f32)
    md_ref[...] = (
        jnp.dot(nd_ref[...], wd_ref[...], preferred_element_type=f32) + bd_ref[...]
    )


def _node_msgs(ns_p, nd_p, W_src, W_dst, b_dst):
    blk = 1264
    return pl.pallas_call(
        _node_msg_body,
        grid=(NP // blk,),
        in_specs=[
            pl.BlockSpec((blk, D), lambda i: (i, 0)),
            pl.BlockSpec((blk, D), lambda i: (i, 0)),
            pl.BlockSpec((D, D), lambda i: (0, 0)),
            pl.BlockSpec((D, D), lambda i: (0, 0)),
            pl.BlockSpec((1, D), lambda i: (0, 0)),
        ],
        out_specs=[
            pl.BlockSpec((blk, D), lambda i: (i, 0)),
            pl.BlockSpec((blk, D), lambda i: (i, 0)),
        ],
        out_shape=[
            jax.ShapeDtypeStruct((NP, D), f32),
            jax.ShapeDtypeStruct((NP, D), f32),
        ],
    )(ns_p, nd_p, W_src, W_dst, b_dst.reshape(1, D))


def _sc_gather(msrc, mdst, esrc2d, edst2d):
    mesh = plsc.VectorSubcoreMesh(core_axis_name="c", subcore_axis_name="s")

    @functools.partial(
        pl.kernel,
        mesh=mesh,
        out_type=(
            jax.ShapeDtypeStruct((EP, D), f32),
            jax.ShapeDtypeStruct((EP, D), f32),
        ),
        scratch_types=[
            pltpu.VMEM((NCH, GC), jnp.int32),
            pltpu.VMEM((NCH, GC), jnp.int32),
            pltpu.VMEM((3, GC, D), f32),
            pltpu.VMEM((3, GC, D), f32),
            pltpu.SemaphoreType.DMA,
            pltpu.SemaphoreType.DMA,
            pltpu.SemaphoreType.DMA,
            pltpu.SemaphoreType.DMA,
            pltpu.SemaphoreType.DMA,
            pltpu.SemaphoreType.DMA,
        ],
    )
    def k(msrc_h, mdst_h, esrc_h, edst_h, outs_h, outd_h,
          ixs, ixd, ra, rb, g0, g1, g2, w0, w1, w2):
        c = lax.axis_index("c")
        s = lax.axis_index("s")
        wid = s * 2 + c
        chunk0 = wid * NCH
        base0 = wid * EPW
        gsem = (g0, g1, g2)
        wsem = (w0, w1, w2)
        pltpu.sync_copy(esrc_h.at[pl.ds(chunk0, NCH), :], ixs)
        pltpu.sync_copy(edst_h.at[pl.ds(chunk0, NCH), :], ixd)

        def g_start(i, b):
            pltpu.make_async_copy(msrc_h.at[ixs.at[i]], ra.at[b], gsem[b]).start()
            pltpu.make_async_copy(mdst_h.at[ixd.at[i]], rb.at[b], gsem[b]).start()

        def g_wait(i, b):
            pltpu.make_async_copy(msrc_h.at[ixs.at[i]], ra.at[b], gsem[b]).wait()
            pltpu.make_async_copy(mdst_h.at[ixd.at[i]], rb.at[b], gsem[b]).wait()

        def w_start(i, b):
            base = base0 + i * GC
            pltpu.make_async_copy(ra.at[b], outs_h.at[pl.ds(base, GC), :], wsem[b]).start()
            pltpu.make_async_copy(rb.at[b], outd_h.at[pl.ds(base, GC), :], wsem[b]).start()

        def w_wait(b):
            pltpu.make_async_copy(ra.at[b], outs_h.at[pl.ds(0, GC), :], wsem[b]).wait()
            pltpu.make_async_copy(rb.at[b], outd_h.at[pl.ds(0, GC), :], wsem[b]).wait()

        g_start(0, 0)
        g_start(1, 1)

        def outer(t, carry):
            for j in range(3):
                i = t * 3 + j
                b = j
                bprev = (j + 2) % 3
                if j == 0:
                    @pl.when(t > 0)
                    def _():
                        w_wait(bprev)
                        g_start(i + 2, bprev)

                    @pl.when(t == 0)
                    def _():
                        g_start(i + 2, bprev)
                else:
                    w_wait(bprev)
                    g_start(i + 2, bprev)
                g_wait(i, b)
                w_start(i, b)
            return carry

        lax.fori_loop(0, NCH // 3, outer, 0)
        w_wait(2)
        g_wait(NCH - 2, 0)
        w_start(NCH - 2, 0)
        w_wait(0)
        g_wait(NCH - 1, 1)
        w_start(NCH - 1, 1)
        w_wait(1)

    return k(msrc, mdst, esrc2d, edst2d)


def _edge_body(ms_ref, md_ref, es_ref, ea_ref, w1_ref, b1_ref, w2_ref, b2_ref,
               wea_ref, aa_ref, rr_ref, wv_ref, ev_ref, ex_ref):
    msg = ms_ref[...] + md_ref[...]
    h = jax.nn.silu(
        jnp.dot(es_ref[...], w1_ref[...], preferred_element_type=f32) + b1_ref[...]
    )
    w = jnp.dot(h, w2_ref[...], preferred_element_type=f32) + b2_ref[...]
    dtp = msg * w + jnp.dot(ea_ref[...], wea_ref[...], preferred_element_type=f32)
    a = jnp.where(dtp >= 0, dtp, 0.2 * dtp)
    exv = jnp.exp(jnp.dot(a, aa_ref[...], preferred_element_type=f32))
    v = jnp.dot(dtp, wv_ref[...], preferred_element_type=f32)
    exw = jnp.dot(exv, rr_ref[...], preferred_element_type=f32)
    ev_ref[...] = v * exw
    ex_ref[...] = exw


def _edge_pipeline(msgg_s, msgg_d, es_p, ea_p, W1, b1, W2, b2, W_ea, Aa, Rr, W_v):
    bt = 512
    return pl.pallas_call(
        _edge_body,
        grid=(EP // bt,),
        in_specs=[
            pl.BlockSpec((bt, D), lambda i: (i, 0)),
            pl.BlockSpec((bt, D), lambda i: (i, 0)),
            pl.BlockSpec((bt, FC), lambda i: (i, 0)),
            pl.BlockSpec((bt, DE), lambda i: (i, 0)),
            pl.BlockSpec((FC, FC), lambda i: (0, 0)),
            pl.BlockSpec((1, FC), lambda i: (0, 0)),
            pl.BlockSpec((FC, D), lambda i: (0, 0)),
            pl.BlockSpec((1, D), lambda i: (0, 0)),
            pl.BlockSpec((DE, D), lambda i: (0, 0)),
            pl.BlockSpec((D, H), lambda i: (0, 0)),
            pl.BlockSpec((H, D), lambda i: (0, 0)),
            pl.BlockSpec((D, D), lambda i: (0, 0)),
        ],
        out_specs=[
            pl.BlockSpec((bt, D), lambda i: (i, 0)),
            pl.BlockSpec((bt, D), lambda i: (i, 0)),
        ],
        out_shape=[
            jax.ShapeDtypeStruct((EP, D), f32),
            jax.ShapeDtypeStruct((EP, D), f32),
        ],
    )(msgg_s, msgg_d, es_p, ea_p, W1, b1.reshape(1, FC), W2, b2.reshape(1, D),
      W_ea, Aa, Rr, W_v)


def _sc_scatter128(vals, edst2d, zS):
    mesh = plsc.VectorSubcoreMesh(core_axis_name="c", subcore_axis_name="s")

    @functools.partial(
        pl.kernel,
        mesh=mesh,
        out_type=jax.ShapeDtypeStruct((2, NP, D), f32),
        scratch_types=[
            pltpu.VMEM((NCH, GC), jnp.int32),
            pltpu.VMEM((2, GC, D), f32),
            pltpu.VMEM_SHARED((NP, D), f32),
            pltpu.SemaphoreType.DMA,
            pltpu.SemaphoreType.DMA,
        ],
    )
    def k(val_h, edst_h, zs_h, out_h, ix, vv, ssh, r0sem, r1sem):
        c = lax.axis_index("c")
        s = lax.axis_index("s")
        rsem = (r0sem, r1sem)
        r0 = s * RPT
        pltpu.sync_copy(zs_h.at[pl.ds(r0, RPT), :], ssh.at[pl.ds(r0, RPT), :])

        base0 = c * (EP // 2) + s * EPW
        chunk0 = c * (EP // (2 * GC)) + s * NCH
        pltpu.sync_copy(edst_h.at[pl.ds(chunk0, NCH), :], ix)
        plsc.subcore_barrier()

        def outer(t, carry):
            i0 = t * 2
            for b in range(2):
                i = i0 + b
                base = base0 + i * GC
                pltpu.make_async_copy(
                    val_h.at[pl.ds(base, GC), :], vv.at[b], rsem[b]).start()
            for b in range(2):
                i = i0 + b
                base = base0 + i * GC
                pltpu.make_async_copy(
                    val_h.at[pl.ds(base, GC), :], vv.at[b], rsem[b]).wait()
                pltpu.sync_copy(vv.at[b], ssh.at[ix.at[i]], add=True)
            return carry

        lax.fori_loop(0, NCH // 2, outer, 0)
        plsc.subcore_barrier()
        pltpu.sync_copy(ssh.at[pl.ds(r0, RPT), :], out_h.at[c, pl.ds(r0, RPT), :])

    return k(vals, edst2d, zS)


def _final_body(s_ref, dn_ref, nd_ref, wp_ref, bp_ref, lnw_ref, lnb_ref,
                wf1_ref, bf1_ref, wf2_ref, bf2_ref, out_ref):
    sv = s_ref[0] + s_ref[1]
    denw = dn_ref[0] + dn_ref[1]
    agg = sv / (denw + 1e-9)
    nf = jnp.dot(agg, wp_ref[...], preferred_element_type=f32) + bp_ref[...]
    node_out = nd_ref[...] + nf
    mu = jnp.mean(node_out, axis=-1, keepdims=True)
    xc = node_out - mu
    var = jnp.mean(xc * xc, axis=-1, keepdims=True)
    nrm = xc / jnp.sqrt(var + 1e-5) * lnw_ref[...] + lnb_ref[...]
    h2 = jax.nn.silu(
        jnp.dot(nrm, wf1_ref[...], preferred_element_type=f32) + bf1_ref[...]
    )
    nf2 = jnp.dot(h2, wf2_ref[...], preferred_element_type=f32) + bf2_ref[...]
    out_ref[...] = node_out + nf2


def _node_final(outS, outDen, node_input_dst, W_proj, b_proj, ln_w, ln_b,
                Wf1, bf1, Wf2, bf2):
    bn = 1000
    return pl.pallas_call(
        _final_body,
        grid=(N // bn,),
        in_specs=[
            pl.BlockSpec((2, bn, D), lambda i: (0, i, 0)),
            pl.BlockSpec((2, bn, D), lambda i: (0, i, 0)),
            pl.BlockSpec((bn, D), lambda i: (i, 0)),
            pl.BlockSpec((D, D), lambda i: (0, 0)),
            pl.BlockSpec((1, D), lambda i: (0, 0)),
            pl.BlockSpec((1, D), lambda i: (0, 0)),
            pl.BlockSpec((1, D), lambda i: (0, 0)),
            pl.BlockSpec((D, DM), lambda i: (0, 0)),
            pl.BlockSpec((1, DM), lambda i: (0, 0)),
            pl.BlockSpec((DM, D), lambda i: (0, 0)),
            pl.BlockSpec((1, D), lambda i: (0, 0)),
        ],
        out_specs=pl.BlockSpec((bn, D), lambda i: (i, 0)),
        out_shape=jax.ShapeDtypeStruct((N, D), f32),
    )(outS, outDen, node_input_dst, W_proj, b_proj.reshape(1, D),
      ln_w.reshape(1, D), ln_b.reshape(1, D), Wf1, bf1.reshape(1, DM),
      Wf2, bf2.reshape(1, D))


def kernel(node_input_src, node_input_dst, batch_dst, edge_src, edge_dst,
           edge_attr, edge_scalars, W_src, W_dst, b_dst, W1, b1, W2, b2, W_ea,
           w_alpha, W_v, W_proj, b_proj, ln_w, ln_b, Wf1, bf1, Wf2, bf2):
    ns_p = jnp.zeros((NP, D), f32).at[:N].set(node_input_src)
    nd_p = jnp.zeros((NP, D), f32).at[:N].set(node_input_dst)
    esrc_p = jnp.concatenate([edge_src, jnp.zeros((EP - E,), jnp.int32)])
    edst_p = jnp.concatenate([edge_dst, jnp.full((EP - E,), N, jnp.int32)])
    es_p = jnp.zeros((EP, FC), f32).at[:E].set(edge_scalars)
    ea_p = jnp.zeros((EP, DE), f32).at[:E].set(edge_attr)
    eye16 = jnp.eye(H, dtype=f32)
    Rr = jnp.repeat(eye16, DH, axis=1)
    Aa = (w_alpha[:, :, None] * eye16[:, None, :]).reshape(D, H)

    msg_src, msg_dst = _node_msgs(ns_p, nd_p, W_src, W_dst, b_dst)
    msgg_s, msgg_d = _sc_gather(msg_src, msg_dst, esrc_p.reshape(EP // GC, GC),
                                edst_p.reshape(EP // GC, GC))
    ev, exw = _edge_pipeline(msgg_s, msgg_d, es_p, ea_p, W1, b1, W2, b2, W_ea,
                             Aa, Rr, W_v)
    zS = jnp.zeros((NP, D), f32)
    edst2d = edst_p.reshape(EP // GC, GC)
    outS = _sc_scatter128(ev, edst2d, zS)
    outDen = _sc_scatter128(exw, edst2d, zS)
    return _node_final(outS, outDen, node_input_dst, W_proj, b_proj, ln_w, ln_b,
                       Wf1, bf1, Wf2, bf2)
